# baseline (speedup 1.0000x reference)
import functools

import jax
import jax.numpy as jnp
from jax import lax
from jax.experimental import pallas as pl
from jax.experimental.pallas import tpu as pltpu
from jax.experimental.pallas import tpu_sc as plsc


def _shift(x, k, axis):
    n = x.shape[axis]
    if axis == 0:
        z = jnp.zeros((abs(k), x.shape[1]), x.dtype)
        if k > 0:
            return jnp.concatenate([z, x[: n - k, :]], axis=0)
        return jnp.concatenate([x[-k:, :], z], axis=0)
    z = jnp.zeros((x.shape[0], abs(k)), x.dtype)
    if k > 0:
        return jnp.concatenate([z, x[:, : n - k]], axis=1)
    return jnp.concatenate([x[:, -k:], z], axis=1)


def _shift_max(x, r, axis):
    fwd, bwd = x, x
    covered = 1
    while covered < r + 1:
        s = min(covered, r + 1 - covered)
        fwd = jnp.maximum(fwd, _shift(fwd, -s, axis))
        bwd = jnp.maximum(bwd, _shift(bwd, s, axis))
        covered += s
    return jnp.maximum(fwd, bwd)


def _maxpool2d(x, r):
    return _shift_max(_shift_max(x, r, 0), r, 1)


def _fused_kernel(img_ref, c_ref, seg_ref, o_ref,
                  enh_ref, segment_ref, confo_ref, confc_ref, mnt_ref):
    kb = img_ref.shape[0]
    for k in range(kb):
        img = img_ref[k]
        mi = jnp.min(img)
        mx = jnp.max(img)
        enh_ref[k] = (img - mi) / (mx - mi + 1e-6)

        seg5 = _maxpool2d((seg_ref[k] > 0.5).astype(jnp.float32), 2)
        segment_ref[k] = seg5

        cm = c_ref[k] * seg5
        local_max = _maxpool2d(cm, 7)
        keep = (cm >= local_max) & (cm > 0.45)
        confc_ref[k] = jnp.where(keep, cm, 0.0)
        confo_ref[k] = jnp.where(keep, o_ref[k], 0.0)
        mnt_ref[k] = jnp.sum(keep.astype(jnp.int32), axis=(0, 1),
                             keepdims=True)


def _sc_segbig(seg_hbm, c_hbm, w_hbm, h_hbm, o_hbm,
               segbig_hbm, conf_hbm, seg_v, bin_v, ver_v,
               buf0, buf1, sem0, sem1, sem2):
    wid = lax.axis_index("s") * 2 + lax.axis_index("c")
    zero16 = jnp.zeros((16,), jnp.float32)
    iota = lax.iota(jnp.int32, 16)

    for k in range(2):
        for j, src in enumerate((c_hbm, w_hbm, h_hbm, o_hbm)):
            pltpu.async_copy(src.at[wid * 2 + k],
                             conf_hbm.at[wid * 2 + k, j], sem2)

    def _zv(j, carry):
        ver_v[pl.ds(j * 16, 16)] = zero16
        return carry
    lax.fori_loop(0, (64 * 72) // 16, _zv, 0)

    def _zb(j, carry):
        bin_v[pl.ds(j * 16, 16)] = zero16
        bin_v[pl.ds(66 * 64 + j * 16, 16)] = zero16
        return carry
    lax.fori_loop(0, 8, _zb, 0)

    def image_body(k, carry):
        i = wid * 2 + k
        pltpu.sync_copy(seg_hbm.at[i], seg_v)

        def _bin(j, c2_):
            v = seg_v[pl.ds(j * 16, 16)]
            bin_v[pl.ds(128 + j * 16, 16)] = jnp.where(v > 0.5, 1.0, 0.0)
            return c2_
        lax.fori_loop(0, 256, _bin, 0)

        def _vert(r, c2_):
            for c in range(4):
                off = r * 64 + c * 16
                acc = bin_v[pl.ds(off, 16)]
                for drow in range(1, 5):
                    acc = jnp.maximum(acc, bin_v[pl.ds(off + drow * 64, 16)])
                ver_v[pl.ds(r * 72 + 2 + c * 16, 16)] = acc
            return c2_
        lax.fori_loop(0, 64, _vert, 0)

        def _block_pair(t, c2_):
            for b, (buf, sem) in enumerate(((buf0, sem0), (buf1, sem1))):
                blk = t * 2 + b

                @pl.when((t > 0) | (k > 0))
                def _wait_prev():
                    pltpu.make_async_copy(
                        buf, segbig_hbm.at[i, pl.ds(0, 64)], sem).wait()

                def _row(p, c3_):
                    r = blk * 8 + p
                    for c in range(4):
                        off = r * 72 + c * 16
                        acc = ver_v[pl.ds(off, 16)]
                        for s in range(1, 5):
                            acc = jnp.maximum(acc, ver_v[pl.ds(off + s, 16)])
                        for q in range(8):
                            idxq = (iota >> 3) + 2 * q
                            v = lax.gather(
                                acc, idxq[:, None],
                                dimension_numbers=lax.GatherDimensionNumbers(
                                    offset_dims=(),
                                    collapsed_slice_dims=(0,),
                                    start_index_map=(0,)),
                                slice_sizes=(1,),
                                mode=lax.GatherScatterMode.PROMISE_IN_BOUNDS)
                            for p2 in range(8):
                                buf[p * 8 + p2,
                                    pl.ds((c * 8 + q) * 16, 16)] = v
                    return c3_
                lax.fori_loop(0, 8, _row, 0)

                pltpu.async_copy(
                    buf, segbig_hbm.at[i, pl.ds(blk * 64, 64)], sem)
            return c2_
        lax.fori_loop(0, 4, _block_pair, 0)
        return carry

    lax.fori_loop(0, 2, image_body, 0)

    i_last = wid * 2 + 1
    pltpu.make_async_copy(buf0, segbig_hbm.at[i_last, pl.ds(0, 64)],
                          sem0).wait()
    pltpu.make_async_copy(buf1, segbig_hbm.at[i_last, pl.ds(0, 64)],
                          sem1).wait()
    for k in range(2):
        for j, src in enumerate((c_hbm, w_hbm, h_hbm, o_hbm)):
            pltpu.make_async_copy(src.at[wid * 2 + k],
                                  conf_hbm.at[wid * 2 + k, j], sem2).wait()


def _make_sc_call(B, H, W, Hm, Wm):
    mesh = plsc.VectorSubcoreMesh(core_axis_name="c", subcore_axis_name="s")
    return pl.kernel(
        _sc_segbig,
        mesh=mesh,
        out_type=[jax.ShapeDtypeStruct((B, H, W), jnp.float32),
                  jax.ShapeDtypeStruct((B, 4, Hm * Wm), jnp.float32)],
        scratch_types=[
            pltpu.VMEM((Hm * Wm,), jnp.float32),
            pltpu.VMEM(((Hm + 4) * Wm,), jnp.float32),
            pltpu.VMEM((Hm * (Wm + 8),), jnp.float32),
            pltpu.VMEM((64, W), jnp.float32),
            pltpu.VMEM((64, W), jnp.float32),
            pltpu.SemaphoreType.DMA,
            pltpu.SemaphoreType.DMA,
            pltpu.SemaphoreType.DMA,
        ],
    )


@functools.partial(jax.jit, static_argnames=())
def kernel(imgs, c, seg, w, h, o):
    B = imgs.shape[0]
    H, W = imgs.shape[2], imgs.shape[3]
    Hm, Wm = c.shape[2], c.shape[3]

    img3 = imgs.reshape(B, H, W)
    maps = [x.reshape(B, Hm, Wm) for x in (c, seg, o)]

    segbig, conf = _make_sc_call(B, H, W, Hm, Wm)(
        seg.reshape(B, Hm * Wm), c.reshape(B, Hm * Wm),
        w.reshape(B, Hm * Wm), h.reshape(B, Hm * Wm),
        o.reshape(B, Hm * Wm))

    KB = 8
    big_spec = pl.BlockSpec((KB, H, W), lambda b: (b, 0, 0))
    map_spec = pl.BlockSpec((KB, Hm, Wm), lambda b: (b, 0, 0))

    outs = pl.pallas_call(
        _fused_kernel,
        grid=(B // KB,),
        in_specs=[big_spec] + [map_spec] * 3,
        out_specs=[
            big_spec,
            map_spec,
            map_spec,
            map_spec,
            pl.BlockSpec((KB, 1, 1), lambda b: (b, 0, 0)),
        ],
        out_shape=[
            jax.ShapeDtypeStruct((B, H, W), jnp.float32),
            jax.ShapeDtypeStruct((B, Hm, Wm), jnp.float32),
            jax.ShapeDtypeStruct((B, Hm, Wm), jnp.float32),
            jax.ShapeDtypeStruct((B, Hm, Wm), jnp.float32),
            jax.ShapeDtypeStruct((B, 1, 1), jnp.int32),
        ],
        compiler_params=pltpu.CompilerParams(
            dimension_semantics=("parallel",)),
    )(img3, *maps)

    enh, segment, confo, confc, mnt = outs
    return (enh.reshape(B, 1, H, W),
            segment.reshape(B, 1, Hm, Wm),
            segbig.reshape(B, 1, H, W),
            confo.reshape(B, 1, Hm, Wm),
            confc.reshape(B, 1, Hm, Wm),
            mnt.reshape(B),
            conf.reshape(B, 4, Hm, Wm))


# speedup vs baseline: 1.7818x; 1.7818x over previous
import functools

import jax
import jax.numpy as jnp
from jax import lax
from jax.experimental import pallas as pl
from jax.experimental.pallas import tpu as pltpu
from jax.experimental.pallas import tpu_sc as plsc


def _shift(x, k, axis):
    n = x.shape[axis]
    if axis == 0:
        z = jnp.zeros((abs(k), x.shape[1]), x.dtype)
        if k > 0:
            return jnp.concatenate([z, x[: n - k, :]], axis=0)
        return jnp.concatenate([x[-k:, :], z], axis=0)
    z = jnp.zeros((x.shape[0], abs(k)), x.dtype)
    if k > 0:
        return jnp.concatenate([z, x[:, : n - k]], axis=1)
    return jnp.concatenate([x[:, -k:], z], axis=1)


def _shift_max(x, r, axis):
    fwd, bwd = x, x
    covered = 1
    while covered < r + 1:
        s = min(covered, r + 1 - covered)
        fwd = jnp.maximum(fwd, _shift(fwd, -s, axis))
        bwd = jnp.maximum(bwd, _shift(bwd, s, axis))
        covered += s
    return jnp.maximum(fwd, bwd)


def _maxpool2d(x, r):
    return _shift_max(_shift_max(x, r, 0), r, 1)


def _fused_kernel(img_ref, c_ref, seg_ref, o_ref, w_ref, h_ref,
                  enh_ref, segment_ref, confo_ref, confc_ref,
                  mnt_ref, conf_ref):
    kb = img_ref.shape[0]
    for k in range(kb):
        img = img_ref[k]
        mi = jnp.min(img)
        mx = jnp.max(img)
        enh_ref[k] = (img - mi) / (mx - mi + 1e-6)

        seg5 = _maxpool2d((seg_ref[k] > 0.5).astype(jnp.float32), 2)
        segment_ref[k] = seg5

        cm = c_ref[k] * seg5
        local_max = _maxpool2d(cm, 7)
        keep = (cm >= local_max) & (cm > 0.45)
        confc_ref[k] = jnp.where(keep, cm, 0.0)
        confo_ref[k] = jnp.where(keep, o_ref[k], 0.0)
        mnt_ref[k] = jnp.sum(keep.astype(jnp.int32), axis=(0, 1),
                             keepdims=True)
        conf_ref[k, 0] = c_ref[k]
        conf_ref[k, 1] = w_ref[k]
        conf_ref[k, 2] = h_ref[k]
        conf_ref[k, 3] = o_ref[k]


def _sc_segbig(seg_hbm, segbig_hbm, seg_v, bin_v, ver_v,
               buf0, buf1, sem0, sem1):
    wid = lax.axis_index("s") * 2 + lax.axis_index("c")
    zero16 = jnp.zeros((16,), jnp.float32)
    iota = lax.iota(jnp.int32, 16)

    def _zv(j, carry):
        ver_v[pl.ds(j * 16, 16)] = zero16
        return carry
    lax.fori_loop(0, (64 * 72) // 16, _zv, 0)

    def _zb(j, carry):
        bin_v[pl.ds(j * 16, 16)] = zero16
        bin_v[pl.ds(66 * 64 + j * 16, 16)] = zero16
        return carry
    lax.fori_loop(0, 8, _zb, 0)

    def image_body(k, carry):
        i = wid * 2 + k
        pltpu.sync_copy(seg_hbm.at[i], seg_v)

        def _bin(j, c2_):
            v = seg_v[pl.ds(j * 16, 16)]
            bin_v[pl.ds(128 + j * 16, 16)] = jnp.where(v > 0.5, 1.0, 0.0)
            return c2_
        lax.fori_loop(0, 256, _bin, 0)

        def _vert(r, c2_):
            for c in range(4):
                off = r * 64 + c * 16
                acc = bin_v[pl.ds(off, 16)]
                for drow in range(1, 5):
                    acc = jnp.maximum(acc, bin_v[pl.ds(off + drow * 64, 16)])
                ver_v[pl.ds(r * 72 + 2 + c * 16, 16)] = acc
            return c2_
        lax.fori_loop(0, 64, _vert, 0)

        def _block_pair(t, c2_):
            for b, (buf, sem) in enumerate(((buf0, sem0), (buf1, sem1))):
                blk = t * 2 + b

                @pl.when((t > 0) | (k > 0))
                def _wait_prev():
                    pltpu.make_async_copy(
                        buf, segbig_hbm.at[i, pl.ds(0, 64)], sem).wait()

                def _row(p, c3_):
                    r = blk * 8 + p
                    for c in range(4):
                        off = r * 72 + c * 16
                        acc = ver_v[pl.ds(off, 16)]
                        for s in range(1, 5):
                            acc = jnp.maximum(acc, ver_v[pl.ds(off + s, 16)])
                        for q in range(8):
                            idxq = (iota >> 3) + 2 * q
                            v = lax.gather(
                                acc, idxq[:, None],
                                dimension_numbers=lax.GatherDimensionNumbers(
                                    offset_dims=(),
                                    collapsed_slice_dims=(0,),
                                    start_index_map=(0,)),
                                slice_sizes=(1,),
                                mode=lax.GatherScatterMode.PROMISE_IN_BOUNDS)
                            for p2 in range(8):
                                buf[p * 8 + p2,
                                    pl.ds((c * 8 + q) * 16, 16)] = v
                    return c3_
                lax.fori_loop(0, 8, _row, 0)

                pltpu.async_copy(
                    buf, segbig_hbm.at[i, pl.ds(blk * 64, 64)], sem)
            return c2_
        lax.fori_loop(0, 4, _block_pair, 0)
        return carry

    lax.fori_loop(0, 2, image_body, 0)

    i_last = wid * 2 + 1
    pltpu.make_async_copy(buf0, segbig_hbm.at[i_last, pl.ds(0, 64)],
                          sem0).wait()
    pltpu.make_async_copy(buf1, segbig_hbm.at[i_last, pl.ds(0, 64)],
                          sem1).wait()


def _make_sc_call(B, H, W, Hm, Wm):
    mesh = plsc.VectorSubcoreMesh(core_axis_name="c", subcore_axis_name="s")
    return pl.kernel(
        _sc_segbig,
        mesh=mesh,
        out_type=jax.ShapeDtypeStruct((B, H, W), jnp.float32),
        scratch_types=[
            pltpu.VMEM((Hm * Wm,), jnp.float32),
            pltpu.VMEM(((Hm + 4) * Wm,), jnp.float32),
            pltpu.VMEM((Hm * (Wm + 8),), jnp.float32),
            pltpu.VMEM((64, W), jnp.float32),
            pltpu.VMEM((64, W), jnp.float32),
            pltpu.SemaphoreType.DMA,
            pltpu.SemaphoreType.DMA,
        ],
    )


@functools.partial(jax.jit, static_argnames=())
def kernel(imgs, c, seg, w, h, o):
    B = imgs.shape[0]
    H, W = imgs.shape[2], imgs.shape[3]
    Hm, Wm = c.shape[2], c.shape[3]

    img3 = imgs.reshape(B, H, W)
    maps = [x.reshape(B, Hm, Wm) for x in (c, seg, o, w, h)]

    segbig = _make_sc_call(B, H, W, Hm, Wm)(seg.reshape(B, Hm * Wm))

    KB = 8
    big_spec = pl.BlockSpec((KB, H, W), lambda b: (b, 0, 0))
    map_spec = pl.BlockSpec((KB, Hm, Wm), lambda b: (b, 0, 0))

    outs = pl.pallas_call(
        _fused_kernel,
        grid=(B // KB,),
        in_specs=[big_spec] + [map_spec] * 5,
        out_specs=[
            big_spec,
            map_spec,
            map_spec,
            map_spec,
            pl.BlockSpec((KB, 1, 1), lambda b: (b, 0, 0)),
            pl.BlockSpec((KB, 4, Hm, Wm), lambda b: (b, 0, 0, 0)),
        ],
        out_shape=[
            jax.ShapeDtypeStruct((B, H, W), jnp.float32),
            jax.ShapeDtypeStruct((B, Hm, Wm), jnp.float32),
            jax.ShapeDtypeStruct((B, Hm, Wm), jnp.float32),
            jax.ShapeDtypeStruct((B, Hm, Wm), jnp.float32),
            jax.ShapeDtypeStruct((B, 1, 1), jnp.int32),
            jax.ShapeDtypeStruct((B, 4, Hm, Wm), jnp.float32),
        ],
        compiler_params=pltpu.CompilerParams(
            dimension_semantics=("parallel",)),
    )(img3, *maps)

    enh, segment, confo, confc, mnt, conf = outs
    return (enh.reshape(B, 1, H, W),
            segment.reshape(B, 1, Hm, Wm),
            segbig.reshape(B, 1, H, W),
            confo.reshape(B, 1, Hm, Wm),
            confc.reshape(B, 1, Hm, Wm),
            mnt.reshape(B),
            conf)
